# baseline (speedup 1.0000x reference)
import functools

import jax
import jax.numpy as jnp
from jax import lax
from jax.experimental import pallas as pl
from jax.experimental.pallas import tpu as pltpu
from jax.experimental.pallas import tpu_sc as plsc

N = 10000
E = 320000
D = 128
H = 128
C = 40

NC = 2
NS = 16
NW = NC * NS
K = 128
CPP = 16
NPH0 = 3
NPH1 = 7
EPP = CPP * K
E0 = NS * NPH0 * EPP
E1 = NS * NPH1 * EPP
E_PAD = E0 + E1
NBUF = 2
NP = N + 8
ROWS_PT = 624
TAIL_OFF = ROWS_PT * NS
TAIL_IN = NP - TAIL_OFF
TAIL_OUT = N - TAIL_OFF


@functools.cache
def _get_agg_kernel():
    sc_mesh = plsc.VectorSubcoreMesh(
        core_axis_name="c", subcore_axis_name="s", num_cores=NC, num_subcores=NS
    )
    return pl.kernel(
        _agg_body,
        out_type=jax.ShapeDtypeStruct((NC, N, D), jnp.float32),
        mesh=sc_mesh,
        scratch_types=[
            pltpu.VMEM_SHARED((NP, D), jnp.float32),
            pltpu.VMEM((CPP, K), jnp.int32),
            pltpu.VMEM((CPP, K), jnp.int32),
            pltpu.VMEM((K, D), jnp.float32),
            pltpu.VMEM((K, D), jnp.float32),
            pltpu.SemaphoreType.DMA,
            pltpu.SemaphoreType.DMA,
        ],
    )


def _agg_body(x_hbm, src0_hbm, dst0_hbm, src1_hbm, dst1_hbm, out_hbm,
              acc, idx_s, idx_d, rb0, rb1, sem0, sem1):
    c = lax.axis_index("c")
    s = lax.axis_index("s")
    rows = (rb0, rb1)
    sems = (sem0, sem1)

    pltpu.sync_copy(
        x_hbm.at[pl.ds(s * ROWS_PT, ROWS_PT)],
        acc.at[pl.ds(s * ROWS_PT, ROWS_PT)],
    )

    @pl.when(s == 0)
    def _init_tail():
        pltpu.sync_copy(
            x_hbm.at[pl.ds(TAIL_OFF, TAIL_IN)], acc.at[pl.ds(TAIL_OFF, TAIL_IN)]
        )

    plsc.subcore_barrier()

    def start(b, j):
        pltpu.async_copy(x_hbm.at[idx_s.at[j]], rows[b], sems[b])

    def finish(b, j):
        pltpu.make_async_copy(x_hbm.at[idx_s.at[j]], rows[b], sems[b]).wait()
        pltpu.sync_copy(rows[b], acc.at[idx_d.at[j]], add=True)

    def run_core(src_hbm, dst_hbm, nph):
        @pl.loop(0, nph)
        def _phase(p):
            pltpu.sync_copy(src_hbm.at[s, p], idx_s)
            pltpu.sync_copy(dst_hbm.at[s, p], idx_d)

            for b in range(NBUF):
                start(b, b)

            @pl.loop(0, CPP - NBUF, step=NBUF)
            def _steady(g):
                for b in range(NBUF):
                    finish(b, g + b)
                    start(b, g + b + NBUF)

            for b in range(NBUF):
                finish(b, CPP - NBUF + b)

    @pl.when(c == 0)
    def _core0():
        run_core(src0_hbm, dst0_hbm, NPH0)

    @pl.when(c == 1)
    def _core1():
        run_core(src1_hbm, dst1_hbm, NPH1)

    plsc.subcore_barrier()
    pltpu.sync_copy(
        acc.at[pl.ds(s * ROWS_PT, ROWS_PT)],
        out_hbm.at[c, pl.ds(s * ROWS_PT, ROWS_PT)],
    )

    @pl.when(s == 0)
    def _write_tail():
        pltpu.sync_copy(
            acc.at[pl.ds(TAIL_OFF, TAIL_OUT)],
            out_hbm.at[c, pl.ds(TAIL_OFF, TAIL_OUT)],
        )


def _mlp_body(x_ref, agg_ref, w1_ref, b1_ref, g_ref, be_ref, w2_ref, b2_ref,
              w3_ref, b3_ref, out_ref):
    h0 = agg_ref[0] + agg_ref[1] - x_ref[...]
    h = jnp.dot(h0, w1_ref[...], preferred_element_type=jnp.float32) + b1_ref[...]
    mean = jnp.mean(h, axis=0, keepdims=True)
    var = jnp.mean(jnp.square(h - mean), axis=0, keepdims=True)
    h = (h - mean) / jnp.sqrt(var + 1e-5) * g_ref[...] + be_ref[...]
    h = jnp.maximum(h, 0.0)
    h = jnp.dot(h, w2_ref[...], preferred_element_type=jnp.float32) + b2_ref[...]
    h = jnp.maximum(h, 0.0)
    out_ref[...] = jnp.dot(h, w3_ref[...], preferred_element_type=jnp.float32) + b3_ref[...]


_mlp_call = pl.pallas_call(
    _mlp_body,
    out_shape=jax.ShapeDtypeStruct((N, 128), jnp.float32),
)


def kernel(x, edge_index, W1, b1, gamma, beta, W2, b2, W3, b3):
    ei = edge_index.astype(jnp.int32)
    pad = jnp.full((E_PAD - E,), N, jnp.int32)
    src = jnp.concatenate([ei[0], pad])
    dst = jnp.concatenate([ei[1], pad])
    src0 = src[:E0].reshape(NS, NPH0, CPP, K)
    dst0 = dst[:E0].reshape(NS, NPH0, CPP, K)
    src1 = src[E0:].reshape(NS, NPH1, CPP, K)
    dst1 = dst[E0:].reshape(NS, NPH1, CPP, K)
    x_pad = jnp.concatenate([x, jnp.zeros((NP - N, D), x.dtype)])
    partials = _get_agg_kernel()(x_pad, src0, dst0, src1, dst1)
    w3p = jnp.zeros((H, 128), W3.dtype).at[:, :C].set(W3)
    b3p = jnp.zeros((128,), b3.dtype).at[:C].set(b3)
    out = _mlp_call(
        x, partials, W1,
        b1.reshape(1, H), gamma.reshape(1, H), beta.reshape(1, H),
        W2, b2.reshape(1, H), w3p, b3p.reshape(1, 128),
    )
    return out[:, :C]


# speedup vs baseline: 1.2318x; 1.2318x over previous
import functools

import jax
import jax.numpy as jnp
from jax import lax
from jax.experimental import pallas as pl
from jax.experimental.pallas import tpu as pltpu
from jax.experimental.pallas import tpu_sc as plsc

N = 10000
E = 320000
D = 128
H = 128
C = 40

NC = 2
NS = 16
NW = NC * NS
K = 128
CPP = 32
NPH0 = 2
NPH1 = 3
EPP = CPP * K
E0 = NS * NPH0 * EPP
E1 = NS * NPH1 * EPP
E_PAD = E0 + E1
NBUF = 2
NP = N + 8
ROWS_PT = 624
TAIL_OFF = ROWS_PT * NS
TAIL_IN = NP - TAIL_OFF
TAIL_OUT = N - TAIL_OFF


@functools.cache
def _get_agg_kernel():
    sc_mesh = plsc.VectorSubcoreMesh(
        core_axis_name="c", subcore_axis_name="s", num_cores=NC, num_subcores=NS
    )
    return pl.kernel(
        _agg_body,
        out_type=jax.ShapeDtypeStruct((NC, N, D), jnp.float32),
        mesh=sc_mesh,
        scratch_types=[
            pltpu.VMEM_SHARED((NP, D), jnp.float32),
            pltpu.VMEM((CPP, K), jnp.int32),
            pltpu.VMEM((CPP, K), jnp.int32),
            pltpu.VMEM((K, D), jnp.float32),
            pltpu.VMEM((K, D), jnp.float32),
            pltpu.SemaphoreType.DMA,
            pltpu.SemaphoreType.DMA,
        ],
    )


def _agg_body(x_hbm, src0_hbm, dst0_hbm, src1_hbm, dst1_hbm, out_hbm,
              acc, idx_s, idx_d, rb0, rb1, sem0, sem1):
    c = lax.axis_index("c")
    s = lax.axis_index("s")
    rows = (rb0, rb1)
    sems = (sem0, sem1)

    pltpu.sync_copy(
        x_hbm.at[pl.ds(s * ROWS_PT, ROWS_PT)],
        acc.at[pl.ds(s * ROWS_PT, ROWS_PT)],
    )

    @pl.when(s == 0)
    def _init_tail():
        pltpu.sync_copy(
            x_hbm.at[pl.ds(TAIL_OFF, TAIL_IN)], acc.at[pl.ds(TAIL_OFF, TAIL_IN)]
        )

    plsc.subcore_barrier()

    def start(b, j):
        pltpu.async_copy(x_hbm.at[idx_s.at[j]], rows[b], sems[b])

    def finish(b, j):
        pltpu.make_async_copy(x_hbm.at[idx_s.at[j]], rows[b], sems[b]).wait()
        pltpu.sync_copy(rows[b], acc.at[idx_d.at[j]], add=True)

    def run_core(src_hbm, dst_hbm, nph):
        @pl.loop(0, nph)
        def _phase(p):
            pltpu.sync_copy(src_hbm.at[s, p], idx_s)
            pltpu.sync_copy(dst_hbm.at[s, p], idx_d)

            for b in range(NBUF):
                start(b, b)

            @pl.loop(0, CPP - NBUF, step=NBUF)
            def _steady(g):
                for b in range(NBUF):
                    finish(b, g + b)
                    start(b, g + b + NBUF)

            for b in range(NBUF):
                finish(b, CPP - NBUF + b)

    @pl.when(c == 0)
    def _core0():
        run_core(src0_hbm, dst0_hbm, NPH0)

    @pl.when(c == 1)
    def _core1():
        run_core(src1_hbm, dst1_hbm, NPH1)

    plsc.subcore_barrier()
    pltpu.sync_copy(
        acc.at[pl.ds(s * ROWS_PT, ROWS_PT)],
        out_hbm.at[c, pl.ds(s * ROWS_PT, ROWS_PT)],
    )

    @pl.when(s == 0)
    def _write_tail():
        pltpu.sync_copy(
            acc.at[pl.ds(TAIL_OFF, TAIL_OUT)],
            out_hbm.at[c, pl.ds(TAIL_OFF, TAIL_OUT)],
        )


def _mlp_body(x_ref, agg_ref, w1_ref, b1_ref, g_ref, be_ref, w2_ref, b2_ref,
              w3_ref, b3_ref, out_ref):
    h0 = agg_ref[0] + agg_ref[1] - x_ref[...]
    h = jnp.dot(h0, w1_ref[...], preferred_element_type=jnp.float32) + b1_ref[...]
    mean = jnp.mean(h, axis=0, keepdims=True)
    var = jnp.mean(jnp.square(h - mean), axis=0, keepdims=True)
    h = (h - mean) / jnp.sqrt(var + 1e-5) * g_ref[...] + be_ref[...]
    h = jnp.maximum(h, 0.0)
    h = jnp.dot(h, w2_ref[...], preferred_element_type=jnp.float32) + b2_ref[...]
    h = jnp.maximum(h, 0.0)
    out_ref[...] = jnp.dot(h, w3_ref[...], preferred_element_type=jnp.float32) + b3_ref[...]


_mlp_call = pl.pallas_call(
    _mlp_body,
    out_shape=jax.ShapeDtypeStruct((N, 128), jnp.float32),
)


def kernel(x, edge_index, W1, b1, gamma, beta, W2, b2, W3, b3):
    ei = edge_index.astype(jnp.int32)
    pad = jnp.full((E_PAD - E,), N, jnp.int32)
    src = jnp.concatenate([ei[0], pad])
    dst = jnp.concatenate([ei[1], pad])
    src0 = src[:E0].reshape(NS, NPH0, CPP, K)
    dst0 = dst[:E0].reshape(NS, NPH0, CPP, K)
    src1 = src[E0:].reshape(NS, NPH1, CPP, K)
    dst1 = dst[E0:].reshape(NS, NPH1, CPP, K)
    x_pad = jnp.concatenate([x, jnp.zeros((NP - N, D), x.dtype)])
    partials = _get_agg_kernel()(x_pad, src0, dst0, src1, dst1)
    w3p = jnp.zeros((H, 128), W3.dtype).at[:, :C].set(W3)
    b3p = jnp.zeros((128,), b3.dtype).at[:C].set(b3)
    out = _mlp_call(
        x, partials, W1,
        b1.reshape(1, H), gamma.reshape(1, H), beta.reshape(1, H),
        W2, b2.reshape(1, H), w3p, b3p.reshape(1, 128),
    )
    return out[:, :C]
